# baseline (speedup 1.0000x reference)
import functools

import jax
import jax.numpy as jnp
from jax import lax
from jax.experimental import pallas as pl
from jax.experimental.pallas import tpu as pltpu
from jax.experimental.pallas import tpu_sc as plsc

_W = 128
_SUB = 2
_U = _SUB * _W
_NC = 2
_NS = 16
_NW = _NC * _NS
_LANES = 16


@functools.lru_cache(maxsize=None)
def _make_gather2x(n_rows: int, d: int):
    n_funit = n_rows // _U
    tail = n_rows - n_funit * _U
    n_units = -(-n_rows // _U)
    t_max = -(-n_units // _NW)
    t_pad = -(-(t_max + 8) // 8) * 8
    n_units_pad = max(
        ((w * n_units) // _NW // 8) * 8 + t_pad for w in range(_NW)
    )
    mesh = plsc.VectorSubcoreMesh(
        core_axis_name="c", subcore_axis_name="s",
        num_cores=_NC, num_subcores=_NS,
    )

    n_buf = 3
    lag = n_buf - 2
    t_full = (n_units // _NW) // n_buf * n_buf

    @functools.partial(
        pl.kernel,
        out_type=jax.ShapeDtypeStruct((n_rows, d), jnp.float32),
        mesh=mesh,
        scratch_types=[
            pltpu.VMEM((t_pad, _SUB, _W), jnp.int32),
            pltpu.VMEM((n_buf, _U, d), jnp.float32),
            [pltpu.SemaphoreType.DMA] * n_buf,
            [pltpu.SemaphoreType.DMA] * n_buf,
        ],
    )
    def gather2x(table_hbm, idx_hbm, out_hbm, idx_v, rows_v, sem_g, sem_o):
        wid = lax.axis_index("s") * _NC + lax.axis_index("c")
        g0 = (wid * n_units) // _NW
        cnt = ((wid + 1) * n_units) // _NW - g0
        a0 = pl.multiple_of((g0 // 8) * 8, 8)
        off = g0 - a0
        pltpu.sync_copy(idx_hbm.at[pl.ds(a0, t_pad)], idx_v)

        def start(t, buf):
            for j in range(_SUB):
                pltpu.async_copy(
                    table_hbm.at[idx_v.at[t + off, j]],
                    rows_v.at[buf, pl.ds(j * _W, _W)],
                    sem_g[buf],
                )

        def wait_gather(buf):
            for j in range(_SUB):
                pltpu.make_async_copy(
                    table_hbm.at[pl.ds(0, _W)],
                    rows_v.at[buf, pl.ds(j * _W, _W)],
                    sem_g[buf],
                ).wait()

        def scale(buf):
            @pl.loop(0, _U, unroll=8)
            def _(r):
                for k in range(d // _LANES):
                    sl = pl.ds(k * _LANES, _LANES)
                    v = rows_v[buf, r, sl]
                    rows_v[buf, r, sl] = v + v

        def _write_parts(buf, u, go):
            @pl.when(u < n_funit)
            def _():
                go(rows_v.at[buf], out_hbm.at[pl.ds(u * _U, _U)], sem_o[buf])

            if tail:
                @pl.when(u == n_funit)
                def _():
                    go(
                        rows_v.at[buf, pl.ds(0, tail)],
                        out_hbm.at[pl.ds(n_funit * _U, tail)],
                        sem_o[buf],
                    )

        def flush_async(buf, u):
            _write_parts(buf, u, pltpu.async_copy)

        def drain_write(buf, u):
            _write_parts(
                buf, u, lambda s, dd, sm: pltpu.make_async_copy(s, dd, sm).wait()
            )

        def step(t, buf):
            wait_gather(buf)
            scale(buf)
            flush_async(buf, g0 + t)
            nxt = (buf + 2) % n_buf

            @pl.when(t >= lag)
            def _():
                drain_write(nxt, g0 + t - lag)

            @pl.when(t + 2 < cnt)
            def _():
                start(t + 2, nxt)

        start(0, 0)
        start(1, 1)

        @pl.loop(0, t_full // n_buf)
        def _(p):
            for i in range(n_buf):
                step(p * n_buf + i, i)

        for i in range(n_buf):
            t = t_full + i

            @pl.when(t < cnt)
            def _():
                step(t, t % n_buf)

        for dt in range(lag, 0, -1):
            for b in range(n_buf):
                @pl.when((cnt - dt) % n_buf == b)
                def _():
                    drain_write(b, g0 + cnt - dt)

    return gather2x, n_units_pad


def kernel(memory, source_nodes, timestamps, n_layers, time_w, time_b):
    del timestamps, n_layers, time_w, time_b
    n_rows = source_nodes.shape[0]
    d = memory.shape[1]
    idx = source_nodes.astype(jnp.int32)
    fn, n_units_pad = _make_gather2x(n_rows, d)
    idx3d = jnp.pad(idx, (0, n_units_pad * _U - n_rows)).reshape(
        n_units_pad, _SUB, _W
    )
    return fn(memory, idx3d)


# speedup vs baseline: 1.0307x; 1.0307x over previous
import functools

import jax
import jax.numpy as jnp
from jax import lax
from jax.experimental import pallas as pl
from jax.experimental.pallas import tpu as pltpu
from jax.experimental.pallas import tpu_sc as plsc

_W = 128
_SUB = 1
_U = _SUB * _W
_NC = 2
_NS = 16
_NW = _NC * _NS
_LANES = 16


@functools.lru_cache(maxsize=None)
def _make_gather2x(n_rows: int, d: int):
    n_funit = n_rows // _U
    tail = n_rows - n_funit * _U
    n_units = -(-n_rows // _U)
    t_max = -(-n_units // _NW)
    t_pad = -(-(t_max + 8) // 8) * 8
    n_units_pad = max(
        ((w * n_units) // _NW // 8) * 8 + t_pad for w in range(_NW)
    )
    mesh = plsc.VectorSubcoreMesh(
        core_axis_name="c", subcore_axis_name="s",
        num_cores=_NC, num_subcores=_NS,
    )

    n_buf = 6
    ahead = 3
    lag = n_buf - ahead
    t_full = (n_units // _NW) // n_buf * n_buf

    @functools.partial(
        pl.kernel,
        out_type=jax.ShapeDtypeStruct((n_rows, d), jnp.float32),
        mesh=mesh,
        scratch_types=[
            pltpu.VMEM((t_pad, _SUB, _W), jnp.int32),
            pltpu.VMEM((n_buf, _U, d), jnp.float32),
            [pltpu.SemaphoreType.DMA] * n_buf,
            [pltpu.SemaphoreType.DMA] * n_buf,
        ],
    )
    def gather2x(table_hbm, idx_hbm, out_hbm, idx_v, rows_v, sem_g, sem_o):
        wid = lax.axis_index("s") * _NC + lax.axis_index("c")
        g0 = (wid * n_units) // _NW
        cnt = ((wid + 1) * n_units) // _NW - g0
        a0 = pl.multiple_of((g0 // 8) * 8, 8)
        off = g0 - a0
        pltpu.sync_copy(idx_hbm.at[pl.ds(a0, t_pad)], idx_v)

        def start(t, buf):
            for j in range(_SUB):
                pltpu.async_copy(
                    table_hbm.at[idx_v.at[t + off, j]],
                    rows_v.at[buf, pl.ds(j * _W, _W)],
                    sem_g[buf],
                )

        def wait_gather(buf):
            for j in range(_SUB):
                pltpu.make_async_copy(
                    table_hbm.at[pl.ds(0, _W)],
                    rows_v.at[buf, pl.ds(j * _W, _W)],
                    sem_g[buf],
                ).wait()

        def scale(buf):
            @pl.loop(0, _U, unroll=8)
            def _(r):
                for k in range(d // _LANES):
                    sl = pl.ds(k * _LANES, _LANES)
                    v = rows_v[buf, r, sl]
                    rows_v[buf, r, sl] = v + v

        def _write_parts(buf, u, go):
            @pl.when(u < n_funit)
            def _():
                go(rows_v.at[buf], out_hbm.at[pl.ds(u * _U, _U)], sem_o[buf])

            if tail:
                @pl.when(u == n_funit)
                def _():
                    go(
                        rows_v.at[buf, pl.ds(0, tail)],
                        out_hbm.at[pl.ds(n_funit * _U, tail)],
                        sem_o[buf],
                    )

        def flush_async(buf, u):
            _write_parts(buf, u, pltpu.async_copy)

        def drain_write(buf, u):
            _write_parts(
                buf, u, lambda s, dd, sm: pltpu.make_async_copy(s, dd, sm).wait()
            )

        def step(t, buf):
            wait_gather(buf)
            scale(buf)
            flush_async(buf, g0 + t)
            nxt = (buf + ahead) % n_buf

            @pl.when(t >= lag)
            def _():
                drain_write(nxt, g0 + t - lag)

            @pl.when(t + ahead < cnt)
            def _():
                start(t + ahead, nxt)

        for a in range(ahead):
            start(a, a)

        @pl.loop(0, t_full // n_buf)
        def _(p):
            for i in range(n_buf):
                step(p * n_buf + i, i)

        for i in range(n_buf):
            t = t_full + i

            @pl.when(t < cnt)
            def _():
                step(t, t % n_buf)

        for dt in range(lag, 0, -1):
            for b in range(n_buf):
                @pl.when((cnt - dt) % n_buf == b)
                def _():
                    drain_write(b, g0 + cnt - dt)

    return gather2x, n_units_pad


def kernel(memory, source_nodes, timestamps, n_layers, time_w, time_b):
    del timestamps, n_layers, time_w, time_b
    n_rows = source_nodes.shape[0]
    d = memory.shape[1]
    idx = source_nodes.astype(jnp.int32)
    fn, n_units_pad = _make_gather2x(n_rows, d)
    idx3d = jnp.pad(idx, (0, n_units_pad * _U - n_rows)).reshape(
        n_units_pad, _SUB, _W
    )
    return fn(memory, idx3d)


# speedup vs baseline: 1.0323x; 1.0016x over previous
import functools

import jax
import jax.numpy as jnp
from jax import lax
from jax.experimental import pallas as pl
from jax.experimental.pallas import tpu as pltpu
from jax.experimental.pallas import tpu_sc as plsc

_W = 128
_SUB = 1
_U = _SUB * _W
_NC = 2
_NS = 16
_NW = _NC * _NS
_LANES = 16


@functools.lru_cache(maxsize=None)
def _make_gather2x(n_rows: int, d: int):
    n_funit = n_rows // _U
    tail = n_rows - n_funit * _U
    n_units = -(-n_rows // _U)
    t_max = -(-n_units // _NW)
    t_pad = -(-(t_max + 8) // 8) * 8
    n_units_pad = max(
        ((w * n_units) // _NW // 8) * 8 + t_pad for w in range(_NW)
    )
    mesh = plsc.VectorSubcoreMesh(
        core_axis_name="c", subcore_axis_name="s",
        num_cores=_NC, num_subcores=_NS,
    )

    n_buf = 6
    ahead = 4
    lag = n_buf - ahead
    t_full = (n_units // _NW) // n_buf * n_buf

    @functools.partial(
        pl.kernel,
        out_type=jax.ShapeDtypeStruct((n_rows, d), jnp.float32),
        mesh=mesh,
        scratch_types=[
            pltpu.VMEM((t_pad, _SUB, _W), jnp.int32),
            pltpu.VMEM((n_buf, _U, d), jnp.float32),
            [pltpu.SemaphoreType.DMA] * n_buf,
            [pltpu.SemaphoreType.DMA] * n_buf,
        ],
    )
    def gather2x(table_hbm, idx_hbm, out_hbm, idx_v, rows_v, sem_g, sem_o):
        wid = lax.axis_index("s") * _NC + lax.axis_index("c")
        g0 = (wid * n_units) // _NW
        cnt = ((wid + 1) * n_units) // _NW - g0
        a0 = pl.multiple_of((g0 // 8) * 8, 8)
        off = g0 - a0
        pltpu.sync_copy(idx_hbm.at[pl.ds(a0, t_pad)], idx_v)

        def start(t, buf):
            for j in range(_SUB):
                pltpu.async_copy(
                    table_hbm.at[idx_v.at[t + off, j]],
                    rows_v.at[buf, pl.ds(j * _W, _W)],
                    sem_g[buf],
                )

        def wait_gather(buf):
            for j in range(_SUB):
                pltpu.make_async_copy(
                    table_hbm.at[pl.ds(0, _W)],
                    rows_v.at[buf, pl.ds(j * _W, _W)],
                    sem_g[buf],
                ).wait()

        def scale(buf):
            @pl.loop(0, _U, unroll=8)
            def _(r):
                for k in range(d // _LANES):
                    sl = pl.ds(k * _LANES, _LANES)
                    v = rows_v[buf, r, sl]
                    rows_v[buf, r, sl] = v + v

        def _write_parts(buf, u, go):
            @pl.when(u < n_funit)
            def _():
                go(rows_v.at[buf], out_hbm.at[pl.ds(u * _U, _U)], sem_o[buf])

            if tail:
                @pl.when(u == n_funit)
                def _():
                    go(
                        rows_v.at[buf, pl.ds(0, tail)],
                        out_hbm.at[pl.ds(n_funit * _U, tail)],
                        sem_o[buf],
                    )

        def flush_async(buf, u):
            _write_parts(buf, u, pltpu.async_copy)

        def drain_write(buf, u):
            _write_parts(
                buf, u, lambda s, dd, sm: pltpu.make_async_copy(s, dd, sm).wait()
            )

        def step(t, buf):
            wait_gather(buf)
            scale(buf)
            flush_async(buf, g0 + t)
            nxt = (buf + ahead) % n_buf

            @pl.when(t >= lag)
            def _():
                drain_write(nxt, g0 + t - lag)

            @pl.when(t + ahead < cnt)
            def _():
                start(t + ahead, nxt)

        for a in range(ahead):
            start(a, a)

        @pl.loop(0, t_full // n_buf)
        def _(p):
            for i in range(n_buf):
                step(p * n_buf + i, i)

        for i in range(n_buf):
            t = t_full + i

            @pl.when(t < cnt)
            def _():
                step(t, t % n_buf)

        for dt in range(lag, 0, -1):
            for b in range(n_buf):
                @pl.when((cnt - dt) % n_buf == b)
                def _():
                    drain_write(b, g0 + cnt - dt)

    return gather2x, n_units_pad


def kernel(memory, source_nodes, timestamps, n_layers, time_w, time_b):
    del timestamps, n_layers, time_w, time_b
    n_rows = source_nodes.shape[0]
    d = memory.shape[1]
    idx = source_nodes.astype(jnp.int32)
    fn, n_units_pad = _make_gather2x(n_rows, d)
    idx3d = jnp.pad(idx, (0, n_units_pad * _U - n_rows)).reshape(
        n_units_pad, _SUB, _W
    )
    return fn(memory, idx3d)
